# Initial kernel scaffold; baseline (speedup 1.0000x reference)
import jax
import jax.numpy as jnp
from jax.experimental import pallas as pl


def kernel(inputs, emb_table):
    raise NotImplementedError("write your pallas kernel here")


# speedup vs baseline: 29.8768x; 29.8768x over previous
import jax
import jax.numpy as jnp
from jax.experimental import pallas as pl

NUM_CLASSES = 26
HIDDEN = 128
BLOCK_M = 2048


def _body(x_ref, t_ref, o_ref):
    o_ref[...] = jax.lax.dot(
        x_ref[...], t_ref[...], preferred_element_type=jnp.float32
    ) * (1.0 / NUM_CLASSES)


def kernel(inputs, emb_table):
    B, L, C = inputs.shape
    M = B * L
    x = inputs.reshape(M, C)
    out = pl.pallas_call(
        _body,
        grid=(M // BLOCK_M,),
        in_specs=[
            pl.BlockSpec((BLOCK_M, C), lambda i: (i, 0)),
            pl.BlockSpec((C, HIDDEN), lambda i: (0, 0)),
        ],
        out_specs=pl.BlockSpec((BLOCK_M, HIDDEN), lambda i: (i, 0)),
        out_shape=jax.ShapeDtypeStruct((M, HIDDEN), jnp.float32),
    )(x, emb_table)
    return out.reshape(B, L, HIDDEN)
